# baseline (speedup 1.0000x reference)
import functools

import jax
import jax.numpy as jnp
from jax import lax
from jax.experimental import pallas as pl
from jax.experimental.pallas import tpu as pltpu
from jax.experimental.pallas import tpu_sc as plsc

N_ROWS = 100000
D = 128
W = D // 2
B = 200000

NC = 2
NS = 16
NW = NC * NS

C = 112
CHUNKS = 1792
B_PAD = CHUNKS * C

FAST_C = 0
CHF = 56
CHS = 56
RING = 3


def _sc_score_body(table, n1, n2, n3, pos_out, neg_out,
                   idx1_v, idx2_v, idx3_v, rows_v, pos_s, neg_s, sems):
    c = lax.axis_index("c")
    s = lax.axis_index("s")
    is_fast = c == FAST_C
    nch = jnp.where(is_fast, CHF, CHS)
    row0 = jnp.where(is_fast, s * CHF, 16 * CHF + s * CHS)
    cstart = jnp.minimum(row0, CHUNKS - CHF)
    off = row0 - cstart

    pltpu.sync_copy(n1.at[pl.ds(pl.multiple_of(cstart * C, 16), CHF * C)], idx1_v)
    pltpu.sync_copy(n2.at[pl.ds(pl.multiple_of(cstart * C, 16), CHF * C)], idx2_v)
    pltpu.sync_copy(n3.at[pl.ds(pl.multiple_of(cstart * C, 16), CHF * C)], idx3_v)

    idx_refs = (idx1_v, idx2_v, idx3_v)

    def issue(t, b):
        for k in range(3):
            pltpu.make_async_copy(
                table.at[idx_refs[k].at[pl.ds(pl.multiple_of((off + t) * C, 16), C)]],
                rows_v.at[b, k], sems.at[b, k],
            ).start()

    def drain(b):
        for k in range(3):
            pltpu.make_async_copy(
                table.at[idx_refs[k].at[pl.ds(0, C)]],
                rows_v.at[b, k], sems.at[b, k],
            ).wait()

    issue(0, 0)
    issue(1, 1)

    lane = lax.iota(jnp.int32, 16)
    zeros = jnp.zeros((16,), jnp.float32)
    perms = [(lane ^ m).reshape(16, 1) for m in (1, 2, 4, 8)]
    dnums = lax.GatherDimensionNumbers(
        offset_dims=(), collapsed_slice_dims=(0,), start_index_map=(0,))

    def hsum(v):
        for p in perms:
            v = v + lax.gather(v, p, dnums, slice_sizes=(1,),
                               mode=lax.GatherScatterMode.PROMISE_IN_BOUNDS)
        return v

    def unpack2(w):
        hi = lax.bitcast_convert_type(w & jnp.int32(-65536), jnp.float32)
        lo = lax.bitcast_convert_type(lax.shift_left(w, 16), jnp.float32)
        return hi, lo

    def chunk_body(t, _):
        b = lax.rem(t, RING)

        @pl.when(t + 2 < nch)
        def _():
            issue(t + 2, lax.rem(t + 2, RING))

        drain(b)

        tm = lax.rem(t, CHS)

        def group_body(g, _):
            def edge_body(i, carry):
                accp, accn = carry
                e = g * 16 + i
                vp = zeros
                vn = zeros
                for cc in range(4):
                    sh, sl = unpack2(rows_v[b, 0, e, pl.ds(cc * 16, 16)])
                    ph, pl_ = unpack2(rows_v[b, 1, e, pl.ds(cc * 16, 16)])
                    nh, nl = unpack2(rows_v[b, 2, e, pl.ds(cc * 16, 16)])
                    vp = vp + sh * ph + sl * pl_
                    vn = vn + sh * nh + sl * nl
                m = lane == i
                accp = jnp.where(m, hsum(vp), accp)
                accn = jnp.where(m, hsum(vn), accn)
                return accp, accn

            accp, accn = lax.fori_loop(0, 16, edge_body, (zeros, zeros))
            base = pl.multiple_of(tm * C + g * 16, 16)
            pos_s[pl.ds(base, 16)] = accp
            neg_s[pl.ds(base, 16)] = accn
            return 0

        lax.fori_loop(0, C // 16, group_body, 0)

        @pl.when(t == CHS - 1)
        def _():
            pltpu.sync_copy(
                pos_s, pos_out.at[pl.ds(pl.multiple_of(row0 * C, 16), CHS * C)])
            pltpu.sync_copy(
                neg_s, neg_out.at[pl.ds(pl.multiple_of(row0 * C, 16), CHS * C)])

        return 0

    lax.fori_loop(0, nch, chunk_body, 0)

    if CHF > CHS:
        @pl.when(is_fast)
        def _():
            pltpu.sync_copy(
                pos_s.at[pl.ds(0, (CHF - CHS) * C)],
                pos_out.at[pl.ds(pl.multiple_of((row0 + CHS) * C, 16),
                                 (CHF - CHS) * C)])
            pltpu.sync_copy(
                neg_s.at[pl.ds(0, (CHF - CHS) * C)],
                neg_out.at[pl.ds(pl.multiple_of((row0 + CHS) * C, 16),
                                 (CHF - CHS) * C)])


@jax.jit
def _sc_scores(table, n1, n2, n3):
    mesh = plsc.VectorSubcoreMesh(
        core_axis_name="c", subcore_axis_name="s", num_cores=NC, num_subcores=NS
    )
    f = pl.kernel(
        _sc_score_body,
        out_type=(
            jax.ShapeDtypeStruct((B_PAD,), jnp.float32),
            jax.ShapeDtypeStruct((B_PAD,), jnp.float32),
        ),
        mesh=mesh,
        compiler_params=pltpu.CompilerParams(use_tc_tiling_on_sc=False),
        scratch_types=[
            pltpu.VMEM((CHF * C,), jnp.int32),
            pltpu.VMEM((CHF * C,), jnp.int32),
            pltpu.VMEM((CHF * C,), jnp.int32),
            pltpu.VMEM((RING, 3, C, W), jnp.int32),
            pltpu.VMEM((CHS * C,), jnp.float32),
            pltpu.VMEM((CHS * C,), jnp.float32),
            pltpu.SemaphoreType.DMA((RING, 3)),
        ],
    )
    return f(table, n1, n2, n3)


def _tc_loss_body(ps_ref, ns_ref, loss_ref, psig_ref, nsig_ref):
    p = ps_ref[...]
    n = ns_ref[...]
    psig_ref[...] = jax.nn.sigmoid(p)
    nsig_ref[...] = jax.nn.sigmoid(n)
    pos_sum = jnp.sum(jax.nn.softplus(-p))
    neg_sum = jnp.sum(jax.nn.softplus(n))
    loss_ref[...] = ((pos_sum + neg_sum) * (1.0 / B)).reshape(1, 1)


@jax.jit
def _tc_loss(ps, ns):
    return pl.pallas_call(
        _tc_loss_body,
        out_shape=(
            jax.ShapeDtypeStruct((1, 1), jnp.float32),
            jax.ShapeDtypeStruct(ps.shape, jnp.float32),
            jax.ShapeDtypeStruct(ns.shape, jnp.float32),
        ),
    )(ps, ns)


def kernel(ACWA_embeddings, node_1, node_2, node_2_negative):
    ti = lax.bitcast_convert_type(ACWA_embeddings, jnp.int32)

    def rtne(x):
        return lax.shift_right_arithmetic(
            x + jnp.int32(0x7FFF) + (lax.shift_right_arithmetic(x, 16) & 1), 16)

    packed = ((rtne(ti[:, :W]) & jnp.int32(0xFFFF))
              | lax.shift_left(rtne(ti[:, W:]), 16))

    pad = B_PAD - B

    def prep(idx):
        return jnp.concatenate([idx, jnp.zeros((pad,), idx.dtype)])

    pos_f, neg_f = _sc_scores(
        packed, prep(node_1), prep(node_2), prep(node_2_negative))

    ns_f = jnp.where(lax.iota(jnp.int32, B_PAD) < B, neg_f, -100.0)

    loss, psig, nsig = _tc_loss(pos_f, ns_f)

    return (loss.reshape(()), psig[:B], nsig[:B])


# speedup vs baseline: 1.0381x; 1.0381x over previous
import functools

import jax
import jax.numpy as jnp
from jax import lax
from jax.experimental import pallas as pl
from jax.experimental.pallas import tpu as pltpu
from jax.experimental.pallas import tpu_sc as plsc

N_ROWS = 100000
D = 128
W = D // 2
B = 200000

NC = 2
NS = 16
NW = NC * NS

C = 112
CHUNKS = 1792
B_PAD = CHUNKS * C

FAST_C = 0
CHF = 56
CHS = 56
RING = 3


def _sc_score_body(table, n1, n2, n3, pos_out, neg_out,
                   idx1_v, idx2_v, idx3_v, rows_v, pos_s, neg_s, sems):
    c = lax.axis_index("c")
    s = lax.axis_index("s")
    is_fast = c == FAST_C
    nch = jnp.where(is_fast, CHF, CHS)
    row0 = jnp.where(is_fast, s * CHF, 16 * CHF + s * CHS)
    cstart = jnp.minimum(row0, CHUNKS - CHF)
    off = row0 - cstart

    pltpu.sync_copy(n1.at[pl.ds(pl.multiple_of(cstart * C, 16), CHF * C)], idx1_v)
    pltpu.sync_copy(n2.at[pl.ds(pl.multiple_of(cstart * C, 16), CHF * C)], idx2_v)
    pltpu.sync_copy(n3.at[pl.ds(pl.multiple_of(cstart * C, 16), CHF * C)], idx3_v)

    idx_refs = (idx1_v, idx2_v, idx3_v)

    def issue(t, b):
        for k in range(3):
            pltpu.make_async_copy(
                table.at[idx_refs[k].at[pl.ds(pl.multiple_of((off + t) * C, 16), C)]],
                rows_v.at[b, k], sems.at[b, k],
            ).start()

    def drain(b):
        for k in range(3):
            pltpu.make_async_copy(
                table.at[idx_refs[k].at[pl.ds(0, C)]],
                rows_v.at[b, k], sems.at[b, k],
            ).wait()

    issue(0, 0)
    issue(1, 1)

    lane = lax.iota(jnp.int32, 16)
    zeros = jnp.zeros((16,), jnp.float32)
    perms = [(lane ^ m).reshape(16, 1) for m in (1, 2, 4, 8)]
    dnums = lax.GatherDimensionNumbers(
        offset_dims=(), collapsed_slice_dims=(0,), start_index_map=(0,))

    def hsum(v):
        for p in perms:
            v = v + lax.gather(v, p, dnums, slice_sizes=(1,),
                               mode=lax.GatherScatterMode.PROMISE_IN_BOUNDS)
        return v

    def unpack2(w):
        hi = lax.bitcast_convert_type(w & jnp.int32(-65536), jnp.float32)
        lo = lax.bitcast_convert_type(lax.shift_left(w, 16), jnp.float32)
        return hi, lo

    def chunk_body(t, _):
        b = lax.rem(t, RING)

        @pl.when(t + 2 < nch)
        def _():
            issue(t + 2, lax.rem(t + 2, RING))

        drain(b)

        tm = lax.rem(t, CHS)

        def group_body(g, _):
            def edge_body(i, carry):
                accp, accn = carry
                e = g * 16 + i
                vp = zeros
                vn = zeros
                for cc in range(4):
                    sh, sl = unpack2(rows_v[b, 0, e, pl.ds(cc * 16, 16)])
                    ph, pl_ = unpack2(rows_v[b, 1, e, pl.ds(cc * 16, 16)])
                    nh, nl = unpack2(rows_v[b, 2, e, pl.ds(cc * 16, 16)])
                    vp = vp + sh * ph + sl * pl_
                    vn = vn + sh * nh + sl * nl
                m = lane == i
                accp = jnp.where(m, hsum(vp), accp)
                accn = jnp.where(m, hsum(vn), accn)
                return accp, accn

            accp, accn = lax.fori_loop(0, 16, edge_body, (zeros, zeros))
            base = pl.multiple_of(tm * C + g * 16, 16)
            pos_s[pl.ds(base, 16)] = accp
            neg_s[pl.ds(base, 16)] = accn
            return 0

        lax.fori_loop(0, C // 16, group_body, 0)

        @pl.when(t == CHS - 1)
        def _():
            pltpu.sync_copy(
                pos_s, pos_out.at[pl.ds(pl.multiple_of(row0 * C, 16), CHS * C)])
            pltpu.sync_copy(
                neg_s, neg_out.at[pl.ds(pl.multiple_of(row0 * C, 16), CHS * C)])

        return 0

    lax.fori_loop(0, nch, chunk_body, 0)

    if CHF > CHS:
        @pl.when(is_fast)
        def _():
            pltpu.sync_copy(
                pos_s.at[pl.ds(0, (CHF - CHS) * C)],
                pos_out.at[pl.ds(pl.multiple_of((row0 + CHS) * C, 16),
                                 (CHF - CHS) * C)])
            pltpu.sync_copy(
                neg_s.at[pl.ds(0, (CHF - CHS) * C)],
                neg_out.at[pl.ds(pl.multiple_of((row0 + CHS) * C, 16),
                                 (CHF - CHS) * C)])


@jax.jit
def _sc_scores(table, n1, n2, n3):
    mesh = plsc.VectorSubcoreMesh(
        core_axis_name="c", subcore_axis_name="s", num_cores=NC, num_subcores=NS
    )
    f = pl.kernel(
        _sc_score_body,
        out_type=(
            jax.ShapeDtypeStruct((B_PAD,), jnp.float32),
            jax.ShapeDtypeStruct((B_PAD,), jnp.float32),
        ),
        mesh=mesh,
        compiler_params=pltpu.CompilerParams(use_tc_tiling_on_sc=False),
        scratch_types=[
            pltpu.VMEM((CHF * C,), jnp.int32),
            pltpu.VMEM((CHF * C,), jnp.int32),
            pltpu.VMEM((CHF * C,), jnp.int32),
            pltpu.VMEM((RING, 3, C, W), jnp.int32),
            pltpu.VMEM((CHS * C,), jnp.float32),
            pltpu.VMEM((CHS * C,), jnp.float32),
            pltpu.SemaphoreType.DMA((RING, 3)),
        ],
    )
    return f(table, n1, n2, n3)


def _pack_body(t_ref, out_ref):
    def rtne(x):
        return lax.shift_right_arithmetic(
            x + jnp.int32(0x7FFF) + (lax.shift_right_arithmetic(x, 16) & 1), 16)

    r = rtne(lax.bitcast_convert_type(t_ref[...], jnp.int32))
    out_ref[...] = (r[:, :W] & jnp.int32(0xFFFF)) | lax.shift_left(r[:, W:], 16)


@jax.jit
def _pack_table(table):
    blk = 2000
    return pl.pallas_call(
        _pack_body,
        grid=(N_ROWS // blk,),
        in_specs=[pl.BlockSpec((blk, D), lambda i: (i, 0))],
        out_specs=pl.BlockSpec((blk, W), lambda i: (i, 0)),
        out_shape=jax.ShapeDtypeStruct((N_ROWS, W), jnp.int32),
    )(table)


def _tc_loss_body(ps_ref, ns_ref, loss_ref, psig_ref, nsig_ref):
    p = ps_ref[...]
    n = ns_ref[...]
    psig_ref[...] = jax.nn.sigmoid(p)
    nsig_ref[...] = jax.nn.sigmoid(n)
    pos_sum = jnp.sum(jax.nn.softplus(-p))
    neg_sum = jnp.sum(jax.nn.softplus(n))
    loss_ref[...] = ((pos_sum + neg_sum) * (1.0 / B)).reshape(1, 1)


@jax.jit
def _tc_loss(ps, ns):
    return pl.pallas_call(
        _tc_loss_body,
        out_shape=(
            jax.ShapeDtypeStruct((1, 1), jnp.float32),
            jax.ShapeDtypeStruct(ps.shape, jnp.float32),
            jax.ShapeDtypeStruct(ns.shape, jnp.float32),
        ),
    )(ps, ns)


def kernel(ACWA_embeddings, node_1, node_2, node_2_negative):
    packed = _pack_table(ACWA_embeddings)

    pad = B_PAD - B

    def prep(idx):
        return jnp.concatenate([idx, jnp.zeros((pad,), idx.dtype)])

    pos_f, neg_f = _sc_scores(
        packed, prep(node_1), prep(node_2), prep(node_2_negative))

    ns_f = jnp.where(lax.iota(jnp.int32, B_PAD) < B, neg_f, -100.0)

    loss, psig, nsig = _tc_loss(pos_f, ns_f)

    return (loss.reshape(()), psig[:B], nsig[:B])


# speedup vs baseline: 1.2469x; 1.2011x over previous
import functools

import jax
import jax.numpy as jnp
from jax import lax
from jax.experimental import pallas as pl
from jax.experimental.pallas import tpu as pltpu
from jax.experimental.pallas import tpu_sc as plsc

N_ROWS = 100000
D = 128
W = D // 2
B = 200000

NC = 2
NS = 16
NW = NC * NS

C = 112
CHUNKS = 1792
B_PAD = CHUNKS * C

FAST_C = 0
CHF = 56
CHS = 56
RING = 3


def _sc_score_body(table, n1, n2, n3, pos_out, neg_out,
                   idx1_v, idx2_v, idx3_v, rows_v, pos_s, neg_s, sems):
    c = lax.axis_index("c")
    s = lax.axis_index("s")
    is_fast = c == FAST_C
    nch = jnp.where(is_fast, CHF, CHS)
    row0 = jnp.where(is_fast, s * CHF, 16 * CHF + s * CHS)
    cstart = jnp.minimum(row0, CHUNKS - CHF)
    off = row0 - cstart

    pltpu.sync_copy(n1.at[pl.ds(pl.multiple_of(cstart * C, 16), CHF * C)], idx1_v)
    pltpu.sync_copy(n2.at[pl.ds(pl.multiple_of(cstart * C, 16), CHF * C)], idx2_v)
    pltpu.sync_copy(n3.at[pl.ds(pl.multiple_of(cstart * C, 16), CHF * C)], idx3_v)

    idx_refs = (idx1_v, idx2_v, idx3_v)

    def issue(t, b):
        for k in range(3):
            pltpu.make_async_copy(
                table.at[idx_refs[k].at[pl.ds(pl.multiple_of((off + t) * C, 16), C)]],
                rows_v.at[b, k], sems.at[b, k],
            ).start()

    def drain(b):
        for k in range(3):
            pltpu.make_async_copy(
                table.at[idx_refs[k].at[pl.ds(0, C)]],
                rows_v.at[b, k], sems.at[b, k],
            ).wait()

    issue(0, 0)
    issue(1, 1)

    lane = lax.iota(jnp.int32, 16)
    zeros = jnp.zeros((16,), jnp.float32)
    perms = [(lane ^ m).reshape(16, 1) for m in (1, 2, 4, 8)]
    dnums = lax.GatherDimensionNumbers(
        offset_dims=(), collapsed_slice_dims=(0,), start_index_map=(0,))

    def hsum(v):
        for p in perms:
            v = v + lax.gather(v, p, dnums, slice_sizes=(1,),
                               mode=lax.GatherScatterMode.PROMISE_IN_BOUNDS)
        return v

    def unpack2(w):
        hi = lax.bitcast_convert_type(w & jnp.int32(-65536), jnp.float32)
        lo = lax.bitcast_convert_type(lax.shift_left(w, 16), jnp.float32)
        return hi, lo

    def chunk_body(t, _):
        b = lax.rem(t, RING)

        @pl.when(t + 2 < nch)
        def _():
            issue(t + 2, lax.rem(t + 2, RING))

        drain(b)

        tm = lax.rem(t, CHS)

        def group_body(g, _):
            def edge_body(i, carry):
                accp, accn = carry
                e = g * 16 + i
                vp = zeros
                vn = zeros
                for cc in range(4):
                    sh, sl = unpack2(rows_v[b, 0, e, pl.ds(cc * 16, 16)])
                    ph, pl_ = unpack2(rows_v[b, 1, e, pl.ds(cc * 16, 16)])
                    nh, nl = unpack2(rows_v[b, 2, e, pl.ds(cc * 16, 16)])
                    vp = vp + sh * ph + sl * pl_
                    vn = vn + sh * nh + sl * nl
                m = lane == i
                accp = jnp.where(m, hsum(vp), accp)
                accn = jnp.where(m, hsum(vn), accn)
                return accp, accn

            accp, accn = lax.fori_loop(0, 16, edge_body, (zeros, zeros))
            base = pl.multiple_of(tm * C + g * 16, 16)
            pos_s[pl.ds(base, 16)] = accp
            neg_s[pl.ds(base, 16)] = accn
            return 0

        lax.fori_loop(0, C // 16, group_body, 0)

        @pl.when(t == CHS - 1)
        def _():
            pltpu.sync_copy(
                pos_s, pos_out.at[pl.ds(pl.multiple_of(row0 * C, 16), CHS * C)])
            pltpu.sync_copy(
                neg_s, neg_out.at[pl.ds(pl.multiple_of(row0 * C, 16), CHS * C)])

        return 0

    lax.fori_loop(0, nch, chunk_body, 0)

    if CHF > CHS:
        @pl.when(is_fast)
        def _():
            pltpu.sync_copy(
                pos_s.at[pl.ds(0, (CHF - CHS) * C)],
                pos_out.at[pl.ds(pl.multiple_of((row0 + CHS) * C, 16),
                                 (CHF - CHS) * C)])
            pltpu.sync_copy(
                neg_s.at[pl.ds(0, (CHF - CHS) * C)],
                neg_out.at[pl.ds(pl.multiple_of((row0 + CHS) * C, 16),
                                 (CHF - CHS) * C)])


@jax.jit
def _sc_scores(table, n1, n2, n3):
    mesh = plsc.VectorSubcoreMesh(
        core_axis_name="c", subcore_axis_name="s", num_cores=NC, num_subcores=NS
    )
    f = pl.kernel(
        _sc_score_body,
        out_type=(
            jax.ShapeDtypeStruct((B_PAD,), jnp.float32),
            jax.ShapeDtypeStruct((B_PAD,), jnp.float32),
        ),
        mesh=mesh,
        compiler_params=pltpu.CompilerParams(use_tc_tiling_on_sc=False),
        scratch_types=[
            pltpu.VMEM((CHF * C,), jnp.int32),
            pltpu.VMEM((CHF * C,), jnp.int32),
            pltpu.VMEM((CHF * C,), jnp.int32),
            pltpu.VMEM((RING, 3, C, W), jnp.int32),
            pltpu.VMEM((CHS * C,), jnp.float32),
            pltpu.VMEM((CHS * C,), jnp.float32),
            pltpu.SemaphoreType.DMA((RING, 3)),
        ],
    )
    return f(table, n1, n2, n3)


PBLK = 1000


def _pack_body(a_ref, b_ref, out_ref):
    def rtne(x):
        return lax.shift_right_arithmetic(
            x + jnp.int32(0x7FFF) + (lax.shift_right_arithmetic(x, 16) & 1), 16)

    def packw(r):
        return (r[:, :W] & jnp.int32(0xFFFF)) | lax.shift_left(r[:, W:], 16)

    out_ref[:, :W] = packw(rtne(lax.bitcast_convert_type(a_ref[...], jnp.int32)))
    out_ref[:, W:] = packw(rtne(lax.bitcast_convert_type(b_ref[...], jnp.int32)))


@jax.jit
def _pack_table(table):
    return pl.pallas_call(
        _pack_body,
        grid=(N_ROWS // (2 * PBLK),),
        in_specs=[
            pl.BlockSpec((PBLK, D), lambda i: (2 * i, 0)),
            pl.BlockSpec((PBLK, D), lambda i: (2 * i + 1, 0)),
        ],
        out_specs=pl.BlockSpec((PBLK, D), lambda i: (i, 0)),
        out_shape=jax.ShapeDtypeStruct((N_ROWS // 2, D), jnp.int32),
    )(table, table)


def _tc_loss_body(ps_ref, ns_ref, loss_ref, psig_ref, nsig_ref):
    p = ps_ref[...]
    n = ns_ref[...]
    psig_ref[...] = jax.nn.sigmoid(p)
    nsig_ref[...] = jax.nn.sigmoid(n)
    pos_sum = jnp.sum(jax.nn.softplus(-p))
    neg_sum = jnp.sum(jax.nn.softplus(n))
    loss_ref[...] = ((pos_sum + neg_sum) * (1.0 / B)).reshape(1, 1)


@jax.jit
def _tc_loss(ps, ns):
    return pl.pallas_call(
        _tc_loss_body,
        out_shape=(
            jax.ShapeDtypeStruct((1, 1), jnp.float32),
            jax.ShapeDtypeStruct(ps.shape, jnp.float32),
            jax.ShapeDtypeStruct(ns.shape, jnp.float32),
        ),
    )(ps, ns)


def kernel(ACWA_embeddings, node_1, node_2, node_2_negative):
    packed = _pack_table(ACWA_embeddings).reshape(N_ROWS, W)

    pad = B_PAD - B

    def prep(idx):
        idx = (2 * PBLK * (idx // (2 * PBLK)) + 2 * (idx % PBLK)
               + (idx // PBLK) % 2)
        return jnp.concatenate([idx, jnp.zeros((pad,), idx.dtype)])

    pos_f, neg_f = _sc_scores(
        packed, prep(node_1), prep(node_2), prep(node_2_negative))

    ns_f = jnp.where(lax.iota(jnp.int32, B_PAD) < B, neg_f, -100.0)

    loss, psig, nsig = _tc_loss(pos_f, ns_f)

    return (loss.reshape(()), psig[:B], nsig[:B])


# speedup vs baseline: 1.2500x; 1.0025x over previous
import functools

import jax
import jax.numpy as jnp
from jax import lax
from jax.experimental import pallas as pl
from jax.experimental.pallas import tpu as pltpu
from jax.experimental.pallas import tpu_sc as plsc

N_ROWS = 100000
D = 128
W = D // 2
B = 200000

NC = 2
NS = 16
NW = NC * NS

C = 112
CHUNKS = 1792
B_PAD = CHUNKS * C

FAST_C = 0
CHF = 56
CHS = 56
RING = 4


def _sc_score_body(table, n1, n2, n3, pos_out, neg_out,
                   idx1_v, idx2_v, idx3_v, rows_v, pos_s, neg_s, sems):
    c = lax.axis_index("c")
    s = lax.axis_index("s")
    is_fast = c == FAST_C
    nch = jnp.where(is_fast, CHF, CHS)
    row0 = jnp.where(is_fast, s * CHF, 16 * CHF + s * CHS)
    cstart = jnp.minimum(row0, CHUNKS - CHF)
    off = row0 - cstart

    pltpu.sync_copy(n1.at[pl.ds(pl.multiple_of(cstart * C, 16), CHF * C)], idx1_v)
    pltpu.sync_copy(n2.at[pl.ds(pl.multiple_of(cstart * C, 16), CHF * C)], idx2_v)
    pltpu.sync_copy(n3.at[pl.ds(pl.multiple_of(cstart * C, 16), CHF * C)], idx3_v)

    idx_refs = (idx1_v, idx2_v, idx3_v)

    def issue(t, b):
        for k in range(3):
            pltpu.make_async_copy(
                table.at[idx_refs[k].at[pl.ds(pl.multiple_of((off + t) * C, 16), C)]],
                rows_v.at[b, k], sems.at[b, k],
            ).start()

    def drain(b):
        for k in range(3):
            pltpu.make_async_copy(
                table.at[idx_refs[k].at[pl.ds(0, C)]],
                rows_v.at[b, k], sems.at[b, k],
            ).wait()

    issue(0, 0)
    issue(1, 1)

    lane = lax.iota(jnp.int32, 16)
    zeros = jnp.zeros((16,), jnp.float32)
    perms = [(lane ^ m).reshape(16, 1) for m in (1, 2, 4, 8)]
    dnums = lax.GatherDimensionNumbers(
        offset_dims=(), collapsed_slice_dims=(0,), start_index_map=(0,))

    def hsum(v):
        for p in perms:
            v = v + lax.gather(v, p, dnums, slice_sizes=(1,),
                               mode=lax.GatherScatterMode.PROMISE_IN_BOUNDS)
        return v

    def unpack2(w):
        hi = lax.bitcast_convert_type(w & jnp.int32(-65536), jnp.float32)
        lo = lax.bitcast_convert_type(lax.shift_left(w, 16), jnp.float32)
        return hi, lo

    def chunk_body(t, _):
        b = lax.rem(t, RING)

        @pl.when(t + 2 < nch)
        def _():
            issue(t + 2, lax.rem(t + 2, RING))

        drain(b)

        tm = lax.rem(t, CHS)

        def group_body(g, _):
            def edge_body(i, carry):
                accp, accn = carry
                e = g * 16 + i
                vp = zeros
                vn = zeros
                for cc in range(4):
                    sh, sl = unpack2(rows_v[b, 0, e, pl.ds(cc * 16, 16)])
                    ph, pl_ = unpack2(rows_v[b, 1, e, pl.ds(cc * 16, 16)])
                    nh, nl = unpack2(rows_v[b, 2, e, pl.ds(cc * 16, 16)])
                    vp = vp + sh * ph + sl * pl_
                    vn = vn + sh * nh + sl * nl
                m = lane == i
                accp = jnp.where(m, hsum(vp), accp)
                accn = jnp.where(m, hsum(vn), accn)
                return accp, accn

            accp, accn = lax.fori_loop(0, 16, edge_body, (zeros, zeros))
            base = pl.multiple_of(tm * C + g * 16, 16)
            pos_s[pl.ds(base, 16)] = accp
            neg_s[pl.ds(base, 16)] = accn
            return 0

        lax.fori_loop(0, C // 16, group_body, 0)

        @pl.when(t == CHS - 1)
        def _():
            pltpu.sync_copy(
                pos_s, pos_out.at[pl.ds(pl.multiple_of(row0 * C, 16), CHS * C)])
            pltpu.sync_copy(
                neg_s, neg_out.at[pl.ds(pl.multiple_of(row0 * C, 16), CHS * C)])

        return 0

    lax.fori_loop(0, nch, chunk_body, 0)

    if CHF > CHS:
        @pl.when(is_fast)
        def _():
            pltpu.sync_copy(
                pos_s.at[pl.ds(0, (CHF - CHS) * C)],
                pos_out.at[pl.ds(pl.multiple_of((row0 + CHS) * C, 16),
                                 (CHF - CHS) * C)])
            pltpu.sync_copy(
                neg_s.at[pl.ds(0, (CHF - CHS) * C)],
                neg_out.at[pl.ds(pl.multiple_of((row0 + CHS) * C, 16),
                                 (CHF - CHS) * C)])


@jax.jit
def _sc_scores(table, n1, n2, n3):
    mesh = plsc.VectorSubcoreMesh(
        core_axis_name="c", subcore_axis_name="s", num_cores=NC, num_subcores=NS
    )
    f = pl.kernel(
        _sc_score_body,
        out_type=(
            jax.ShapeDtypeStruct((B_PAD,), jnp.float32),
            jax.ShapeDtypeStruct((B_PAD,), jnp.float32),
        ),
        mesh=mesh,
        compiler_params=pltpu.CompilerParams(use_tc_tiling_on_sc=False),
        scratch_types=[
            pltpu.VMEM((CHF * C,), jnp.int32),
            pltpu.VMEM((CHF * C,), jnp.int32),
            pltpu.VMEM((CHF * C,), jnp.int32),
            pltpu.VMEM((RING, 3, C, W), jnp.int32),
            pltpu.VMEM((CHS * C,), jnp.float32),
            pltpu.VMEM((CHS * C,), jnp.float32),
            pltpu.SemaphoreType.DMA((RING, 3)),
        ],
    )
    return f(table, n1, n2, n3)


PBLK = 1000


def _pack_body(a_ref, b_ref, out_ref):
    def rtne(x):
        return lax.shift_right_arithmetic(
            x + jnp.int32(0x7FFF) + (lax.shift_right_arithmetic(x, 16) & 1), 16)

    def packw(r):
        return (r[:, :W] & jnp.int32(0xFFFF)) | lax.shift_left(r[:, W:], 16)

    out_ref[:, :W] = packw(rtne(lax.bitcast_convert_type(a_ref[...], jnp.int32)))
    out_ref[:, W:] = packw(rtne(lax.bitcast_convert_type(b_ref[...], jnp.int32)))


@jax.jit
def _pack_table(table):
    return pl.pallas_call(
        _pack_body,
        grid=(N_ROWS // (2 * PBLK),),
        in_specs=[
            pl.BlockSpec((PBLK, D), lambda i: (2 * i, 0)),
            pl.BlockSpec((PBLK, D), lambda i: (2 * i + 1, 0)),
        ],
        out_specs=pl.BlockSpec((PBLK, D), lambda i: (i, 0)),
        out_shape=jax.ShapeDtypeStruct((N_ROWS // 2, D), jnp.int32),
    )(table, table)


def _tc_loss_body(ps_ref, ns_ref, loss_ref, psig_ref, nsig_ref):
    p = ps_ref[...]
    n = ns_ref[...]
    psig_ref[...] = jax.nn.sigmoid(p)
    nsig_ref[...] = jax.nn.sigmoid(n)
    pos_sum = jnp.sum(jax.nn.softplus(-p))
    neg_sum = jnp.sum(jax.nn.softplus(n))
    loss_ref[...] = ((pos_sum + neg_sum) * (1.0 / B)).reshape(1, 1)


@jax.jit
def _tc_loss(ps, ns):
    return pl.pallas_call(
        _tc_loss_body,
        out_shape=(
            jax.ShapeDtypeStruct((1, 1), jnp.float32),
            jax.ShapeDtypeStruct(ps.shape, jnp.float32),
            jax.ShapeDtypeStruct(ns.shape, jnp.float32),
        ),
    )(ps, ns)


def kernel(ACWA_embeddings, node_1, node_2, node_2_negative):
    packed = _pack_table(ACWA_embeddings).reshape(N_ROWS, W)

    pad = B_PAD - B

    def prep(idx):
        idx = (2 * PBLK * (idx // (2 * PBLK)) + 2 * (idx % PBLK)
               + (idx // PBLK) % 2)
        return jnp.concatenate([idx, jnp.zeros((pad,), idx.dtype)])

    pos_f, neg_f = _sc_scores(
        packed, prep(node_1), prep(node_2), prep(node_2_negative))

    ns_f = jnp.where(lax.iota(jnp.int32, B_PAD) < B, neg_f, -100.0)

    loss, psig, nsig = _tc_loss(pos_f, ns_f)

    return (loss.reshape(()), psig[:B], nsig[:B])


# speedup vs baseline: 1.2579x; 1.0064x over previous
import functools

import jax
import jax.numpy as jnp
from jax import lax
from jax.experimental import pallas as pl
from jax.experimental.pallas import tpu as pltpu
from jax.experimental.pallas import tpu_sc as plsc

N_ROWS = 100000
D = 128
W = D // 2
B = 200000

NC = 2
NS = 16
NW = NC * NS

C = 112
CHUNKS = 1792
B_PAD = CHUNKS * C

FAST_C = 0
CHF = 56
CHS = 56
RING = 4


def _sc_score_body(table, n1, n2, n3, pos_out, neg_out,
                   idx1_v, idx2_v, idx3_v, rows_v, pos_s, neg_s, sems):
    c = lax.axis_index("c")
    s = lax.axis_index("s")
    is_fast = c == FAST_C
    nch = jnp.where(is_fast, CHF, CHS)
    row0 = jnp.where(is_fast, s * CHF, 16 * CHF + s * CHS)
    cstart = jnp.minimum(row0, CHUNKS - CHF)
    off = row0 - cstart

    pltpu.sync_copy(n1.at[pl.ds(pl.multiple_of(cstart * C, 16), CHF * C)], idx1_v)
    pltpu.sync_copy(n2.at[pl.ds(pl.multiple_of(cstart * C, 16), CHF * C)], idx2_v)
    pltpu.sync_copy(n3.at[pl.ds(pl.multiple_of(cstart * C, 16), CHF * C)], idx3_v)

    idx_refs = (idx1_v, idx2_v, idx3_v)

    def issue(t, b):
        for k in range(3):
            pltpu.make_async_copy(
                table.at[idx_refs[k].at[pl.ds(pl.multiple_of((off + t) * C, 16), C)]],
                rows_v.at[b, k], sems.at[b, k],
            ).start()

    def drain(b):
        for k in range(3):
            pltpu.make_async_copy(
                table.at[idx_refs[k].at[pl.ds(0, C)]],
                rows_v.at[b, k], sems.at[b, k],
            ).wait()

    for tt in range(RING - 1):
        issue(tt, tt)

    lane = lax.iota(jnp.int32, 16)
    zeros = jnp.zeros((16,), jnp.float32)
    perms = [(lane ^ m).reshape(16, 1) for m in (1, 2, 4, 8)]
    dnums = lax.GatherDimensionNumbers(
        offset_dims=(), collapsed_slice_dims=(0,), start_index_map=(0,))

    def hsum(v):
        for p in perms:
            v = v + lax.gather(v, p, dnums, slice_sizes=(1,),
                               mode=lax.GatherScatterMode.PROMISE_IN_BOUNDS)
        return v

    def unpack2(w):
        hi = lax.bitcast_convert_type(w & jnp.int32(-65536), jnp.float32)
        lo = lax.bitcast_convert_type(lax.shift_left(w, 16), jnp.float32)
        return hi, lo

    def chunk_body(t, _):
        b = lax.rem(t, RING)

        @pl.when(t + RING - 1 < nch)
        def _():
            issue(t + RING - 1, lax.rem(t + RING - 1, RING))

        drain(b)

        tm = lax.rem(t, CHS)

        def group_body(g, _):
            def edge_body(i, carry):
                accp, accn = carry
                e = g * 16 + i
                vp = zeros
                vn = zeros
                for cc in range(4):
                    sh, sl = unpack2(rows_v[b, 0, e, pl.ds(cc * 16, 16)])
                    ph, pl_ = unpack2(rows_v[b, 1, e, pl.ds(cc * 16, 16)])
                    nh, nl = unpack2(rows_v[b, 2, e, pl.ds(cc * 16, 16)])
                    vp = vp + sh * ph + sl * pl_
                    vn = vn + sh * nh + sl * nl
                m = lane == i
                accp = jnp.where(m, hsum(vp), accp)
                accn = jnp.where(m, hsum(vn), accn)
                return accp, accn

            accp, accn = lax.fori_loop(0, 16, edge_body, (zeros, zeros))
            base = pl.multiple_of(tm * C + g * 16, 16)
            pos_s[pl.ds(base, 16)] = accp
            neg_s[pl.ds(base, 16)] = accn
            return 0

        lax.fori_loop(0, C // 16, group_body, 0)

        @pl.when(t == CHS - 1)
        def _():
            pltpu.sync_copy(
                pos_s, pos_out.at[pl.ds(pl.multiple_of(row0 * C, 16), CHS * C)])
            pltpu.sync_copy(
                neg_s, neg_out.at[pl.ds(pl.multiple_of(row0 * C, 16), CHS * C)])

        return 0

    lax.fori_loop(0, nch, chunk_body, 0)

    if CHF > CHS:
        @pl.when(is_fast)
        def _():
            pltpu.sync_copy(
                pos_s.at[pl.ds(0, (CHF - CHS) * C)],
                pos_out.at[pl.ds(pl.multiple_of((row0 + CHS) * C, 16),
                                 (CHF - CHS) * C)])
            pltpu.sync_copy(
                neg_s.at[pl.ds(0, (CHF - CHS) * C)],
                neg_out.at[pl.ds(pl.multiple_of((row0 + CHS) * C, 16),
                                 (CHF - CHS) * C)])


@jax.jit
def _sc_scores(table, n1, n2, n3):
    mesh = plsc.VectorSubcoreMesh(
        core_axis_name="c", subcore_axis_name="s", num_cores=NC, num_subcores=NS
    )
    f = pl.kernel(
        _sc_score_body,
        out_type=(
            jax.ShapeDtypeStruct((B_PAD,), jnp.float32),
            jax.ShapeDtypeStruct((B_PAD,), jnp.float32),
        ),
        mesh=mesh,
        compiler_params=pltpu.CompilerParams(use_tc_tiling_on_sc=False),
        scratch_types=[
            pltpu.VMEM((CHF * C,), jnp.int32),
            pltpu.VMEM((CHF * C,), jnp.int32),
            pltpu.VMEM((CHF * C,), jnp.int32),
            pltpu.VMEM((RING, 3, C, W), jnp.int32),
            pltpu.VMEM((CHS * C,), jnp.float32),
            pltpu.VMEM((CHS * C,), jnp.float32),
            pltpu.SemaphoreType.DMA((RING, 3)),
        ],
    )
    return f(table, n1, n2, n3)


PBLK = 1000


def _pack_body(a_ref, b_ref, out_ref):
    def rtne(x):
        return lax.shift_right_arithmetic(
            x + jnp.int32(0x7FFF) + (lax.shift_right_arithmetic(x, 16) & 1), 16)

    def packw(r):
        return (r[:, :W] & jnp.int32(0xFFFF)) | lax.shift_left(r[:, W:], 16)

    out_ref[:, :W] = packw(rtne(lax.bitcast_convert_type(a_ref[...], jnp.int32)))
    out_ref[:, W:] = packw(rtne(lax.bitcast_convert_type(b_ref[...], jnp.int32)))


@jax.jit
def _pack_table(table):
    return pl.pallas_call(
        _pack_body,
        grid=(N_ROWS // (2 * PBLK),),
        in_specs=[
            pl.BlockSpec((PBLK, D), lambda i: (2 * i, 0)),
            pl.BlockSpec((PBLK, D), lambda i: (2 * i + 1, 0)),
        ],
        out_specs=pl.BlockSpec((PBLK, D), lambda i: (i, 0)),
        out_shape=jax.ShapeDtypeStruct((N_ROWS // 2, D), jnp.int32),
    )(table, table)


def _tc_loss_body(ps_ref, ns_ref, loss_ref, psig_ref, nsig_ref):
    p = ps_ref[...]
    n = ns_ref[...]
    psig_ref[...] = jax.nn.sigmoid(p)
    nsig_ref[...] = jax.nn.sigmoid(n)
    pos_sum = jnp.sum(jax.nn.softplus(-p))
    neg_sum = jnp.sum(jax.nn.softplus(n))
    loss_ref[...] = ((pos_sum + neg_sum) * (1.0 / B)).reshape(1, 1)


@jax.jit
def _tc_loss(ps, ns):
    return pl.pallas_call(
        _tc_loss_body,
        out_shape=(
            jax.ShapeDtypeStruct((1, 1), jnp.float32),
            jax.ShapeDtypeStruct(ps.shape, jnp.float32),
            jax.ShapeDtypeStruct(ns.shape, jnp.float32),
        ),
    )(ps, ns)


def kernel(ACWA_embeddings, node_1, node_2, node_2_negative):
    packed = _pack_table(ACWA_embeddings).reshape(N_ROWS, W)

    pad = B_PAD - B

    def prep(idx):
        idx = (2 * PBLK * (idx // (2 * PBLK)) + 2 * (idx % PBLK)
               + (idx // PBLK) % 2)
        return jnp.concatenate([idx, jnp.zeros((pad,), idx.dtype)])

    pos_f, neg_f = _sc_scores(
        packed, prep(node_1), prep(node_2), prep(node_2_negative))

    ns_f = jnp.where(lax.iota(jnp.int32, B_PAD) < B, neg_f, -100.0)

    loss, psig, nsig = _tc_loss(pos_f, ns_f)

    return (loss.reshape(()), psig[:B], nsig[:B])


# speedup vs baseline: 1.3303x; 1.0575x over previous
import functools

import jax
import jax.numpy as jnp
from jax import lax
from jax.experimental import pallas as pl
from jax.experimental.pallas import tpu as pltpu
from jax.experimental.pallas import tpu_sc as plsc

N_ROWS = 100000
D = 128
W = D // 2
B = 200000

NC = 2
NS = 16
NW = NC * NS

C = 112
CHUNKS = 1792
B_PAD = CHUNKS * C

FAST_C = 0
CHF = 64
CHS = 48
RING = 4


def _sc_score_body(table, n1, n2, n3, pos_out, neg_out,
                   idx1_v, idx2_v, idx3_v, rows_v, pos_s, neg_s, sems):
    c = lax.axis_index("c")
    s = lax.axis_index("s")
    is_fast = c == FAST_C
    nch = jnp.where(is_fast, CHF, CHS)
    row0 = jnp.where(is_fast, s * CHF, 16 * CHF + s * CHS)
    cstart = jnp.minimum(row0, CHUNKS - CHF)
    off = row0 - cstart

    pltpu.sync_copy(n1.at[pl.ds(pl.multiple_of(cstart * C, 16), CHF * C)], idx1_v)
    pltpu.sync_copy(n2.at[pl.ds(pl.multiple_of(cstart * C, 16), CHF * C)], idx2_v)
    pltpu.sync_copy(n3.at[pl.ds(pl.multiple_of(cstart * C, 16), CHF * C)], idx3_v)

    idx_refs = (idx1_v, idx2_v, idx3_v)

    def issue(t, b):
        for k in range(3):
            pltpu.make_async_copy(
                table.at[idx_refs[k].at[pl.ds(pl.multiple_of((off + t) * C, 16), C)]],
                rows_v.at[b, k], sems.at[b, k],
            ).start()

    def drain(b):
        for k in range(3):
            pltpu.make_async_copy(
                table.at[idx_refs[k].at[pl.ds(0, C)]],
                rows_v.at[b, k], sems.at[b, k],
            ).wait()

    for tt in range(RING - 1):
        issue(tt, tt)

    lane = lax.iota(jnp.int32, 16)
    zeros = jnp.zeros((16,), jnp.float32)
    perms = [(lane ^ m).reshape(16, 1) for m in (1, 2, 4, 8)]
    dnums = lax.GatherDimensionNumbers(
        offset_dims=(), collapsed_slice_dims=(0,), start_index_map=(0,))

    def hsum(v):
        for p in perms:
            v = v + lax.gather(v, p, dnums, slice_sizes=(1,),
                               mode=lax.GatherScatterMode.PROMISE_IN_BOUNDS)
        return v

    def unpack2(w):
        hi = lax.bitcast_convert_type(w & jnp.int32(-65536), jnp.float32)
        lo = lax.bitcast_convert_type(lax.shift_left(w, 16), jnp.float32)
        return hi, lo

    def chunk_body(t, _):
        b = lax.rem(t, RING)

        @pl.when(t + RING - 1 < nch)
        def _():
            issue(t + RING - 1, lax.rem(t + RING - 1, RING))

        drain(b)

        tm = lax.rem(t, CHS)

        def group_body(g, _):
            def edge_body(i, carry):
                accp, accn = carry
                e = g * 16 + i
                vp = zeros
                vn = zeros
                for cc in range(4):
                    sh, sl = unpack2(rows_v[b, 0, e, pl.ds(cc * 16, 16)])
                    ph, pl_ = unpack2(rows_v[b, 1, e, pl.ds(cc * 16, 16)])
                    nh, nl = unpack2(rows_v[b, 2, e, pl.ds(cc * 16, 16)])
                    vp = vp + sh * ph + sl * pl_
                    vn = vn + sh * nh + sl * nl
                m = lane == i
                accp = jnp.where(m, hsum(vp), accp)
                accn = jnp.where(m, hsum(vn), accn)
                return accp, accn

            accp, accn = lax.fori_loop(0, 16, edge_body, (zeros, zeros))
            base = pl.multiple_of(tm * C + g * 16, 16)
            pos_s[pl.ds(base, 16)] = accp
            neg_s[pl.ds(base, 16)] = accn
            return 0

        lax.fori_loop(0, C // 16, group_body, 0)

        @pl.when(t == CHS - 1)
        def _():
            pltpu.sync_copy(
                pos_s, pos_out.at[pl.ds(pl.multiple_of(row0 * C, 16), CHS * C)])
            pltpu.sync_copy(
                neg_s, neg_out.at[pl.ds(pl.multiple_of(row0 * C, 16), CHS * C)])

        return 0

    lax.fori_loop(0, nch, chunk_body, 0)

    if CHF > CHS:
        @pl.when(is_fast)
        def _():
            pltpu.sync_copy(
                pos_s.at[pl.ds(0, (CHF - CHS) * C)],
                pos_out.at[pl.ds(pl.multiple_of((row0 + CHS) * C, 16),
                                 (CHF - CHS) * C)])
            pltpu.sync_copy(
                neg_s.at[pl.ds(0, (CHF - CHS) * C)],
                neg_out.at[pl.ds(pl.multiple_of((row0 + CHS) * C, 16),
                                 (CHF - CHS) * C)])


@jax.jit
def _sc_scores(table, n1, n2, n3):
    mesh = plsc.VectorSubcoreMesh(
        core_axis_name="c", subcore_axis_name="s", num_cores=NC, num_subcores=NS
    )
    f = pl.kernel(
        _sc_score_body,
        out_type=(
            jax.ShapeDtypeStruct((B_PAD,), jnp.float32),
            jax.ShapeDtypeStruct((B_PAD,), jnp.float32),
        ),
        mesh=mesh,
        compiler_params=pltpu.CompilerParams(use_tc_tiling_on_sc=False),
        scratch_types=[
            pltpu.VMEM((CHF * C,), jnp.int32),
            pltpu.VMEM((CHF * C,), jnp.int32),
            pltpu.VMEM((CHF * C,), jnp.int32),
            pltpu.VMEM((RING, 3, C, W), jnp.int32),
            pltpu.VMEM((CHS * C,), jnp.float32),
            pltpu.VMEM((CHS * C,), jnp.float32),
            pltpu.SemaphoreType.DMA((RING, 3)),
        ],
    )
    return f(table, n1, n2, n3)


PBLK = 1000


def _pack_body(a_ref, b_ref, out_ref):
    def rtne(x):
        return lax.shift_right_arithmetic(
            x + jnp.int32(0x7FFF) + (lax.shift_right_arithmetic(x, 16) & 1), 16)

    def packw(r):
        return (r[:, :W] & jnp.int32(0xFFFF)) | lax.shift_left(r[:, W:], 16)

    out_ref[:, :W] = packw(rtne(lax.bitcast_convert_type(a_ref[...], jnp.int32)))
    out_ref[:, W:] = packw(rtne(lax.bitcast_convert_type(b_ref[...], jnp.int32)))


@jax.jit
def _pack_table(table):
    return pl.pallas_call(
        _pack_body,
        grid=(N_ROWS // (2 * PBLK),),
        in_specs=[
            pl.BlockSpec((PBLK, D), lambda i: (2 * i, 0)),
            pl.BlockSpec((PBLK, D), lambda i: (2 * i + 1, 0)),
        ],
        out_specs=pl.BlockSpec((PBLK, D), lambda i: (i, 0)),
        out_shape=jax.ShapeDtypeStruct((N_ROWS // 2, D), jnp.int32),
    )(table, table)


def _tc_loss_body(ps_ref, ns_ref, loss_ref, psig_ref, nsig_ref):
    p = ps_ref[...]
    n = ns_ref[...]
    psig_ref[...] = jax.nn.sigmoid(p)
    nsig_ref[...] = jax.nn.sigmoid(n)
    pos_sum = jnp.sum(jax.nn.softplus(-p))
    neg_sum = jnp.sum(jax.nn.softplus(n))
    loss_ref[...] = ((pos_sum + neg_sum) * (1.0 / B)).reshape(1, 1)


@jax.jit
def _tc_loss(ps, ns):
    return pl.pallas_call(
        _tc_loss_body,
        out_shape=(
            jax.ShapeDtypeStruct((1, 1), jnp.float32),
            jax.ShapeDtypeStruct(ps.shape, jnp.float32),
            jax.ShapeDtypeStruct(ns.shape, jnp.float32),
        ),
    )(ps, ns)


def kernel(ACWA_embeddings, node_1, node_2, node_2_negative):
    packed = _pack_table(ACWA_embeddings).reshape(N_ROWS, W)

    pad = B_PAD - B

    def prep(idx):
        idx = (2 * PBLK * (idx // (2 * PBLK)) + 2 * (idx % PBLK)
               + (idx // PBLK) % 2)
        return jnp.concatenate([idx, jnp.zeros((pad,), idx.dtype)])

    pos_f, neg_f = _sc_scores(
        packed, prep(node_1), prep(node_2), prep(node_2_negative))

    ns_f = jnp.where(lax.iota(jnp.int32, B_PAD) < B, neg_f, -100.0)

    loss, psig, nsig = _tc_loss(pos_f, ns_f)

    return (loss.reshape(()), psig[:B], nsig[:B])
